# Initial kernel scaffold; baseline (speedup 1.0000x reference)
import jax
import jax.numpy as jnp
from jax.experimental import pallas as pl


def kernel(x, edge_index, gamma, beta, We1, be1, We2, be2, Wmu, bmu, Wvar, bvar, Wd1, bd1, Wd2, bd2, Wd3, bd3):
    raise NotImplementedError("write your pallas kernel here")


# speedup vs baseline: 1.5684x; 1.5684x over previous
import functools

import jax
import jax.numpy as jnp
from jax import lax
from jax.experimental import pallas as pl
from jax.experimental.pallas import tpu as pltpu
from jax.experimental.pallas import tpu_sc as plsc

_N = 10000
_E = 320000
_D = 128
_BIG = 64
_HID = 32
_NC = 2
_NS = 16
_NW = _NC * _NS
_CH = 128
_KP = 80
_EPT = _KP * _CH
_EP = _NW * _EPT
_NACC = 10112
_RPT = _NACC // _NS
_BE = 1024

_f32 = jnp.float32


def _node1_body(x_ref, g_ref, b_ref, a1_ref, b1_ref, be1_ref, t_ref):
    x = x_ref[...]
    mean = jnp.mean(x, axis=0, keepdims=True)
    xc = x - mean
    var = jnp.mean(xc * xc, axis=0, keepdims=True)
    xn = xc * lax.rsqrt(var + 1e-5) * g_ref[...] + b_ref[...]
    p = jnp.dot(xn, a1_ref[...], preferred_element_type=_f32) + be1_ref[...]
    q = jnp.dot(xn, b1_ref[...], preferred_element_type=_f32)
    t_ref[...] = jnp.concatenate([p, q], axis=1)


def _edge1_body(gd_ref, gs_ref, w2_ref, b2_ref, m_ref):
    pre = jnp.maximum(gd_ref[:, :_BIG] + gs_ref[:, _BIG:], 0.0)
    m = jnp.dot(pre, w2_ref[...], preferred_element_type=_f32) + b2_ref[...]
    m = jnp.maximum(m, 0.0)
    m_ref[...] = jnp.concatenate([m, jnp.ones((_BE, _BIG), _f32)], axis=1)


def _edge2_body(gd_ref, gs_ref, w2_ref, b2_ref, w3_ref, b3_ref, m_ref):
    pre = jnp.maximum(gd_ref[:, :_BIG] + gs_ref[:, _BIG:], 0.0)
    t = jnp.dot(pre, w2_ref[...], preferred_element_type=_f32) + b2_ref[...]
    t = jnp.maximum(t, 0.0)
    m_ref[...] = jnp.dot(t, w3_ref[...], preferred_element_type=_f32) + b3_ref[...]


def _node2_body(acc_ref, eps_ref, wmu_ref, bmu_ref, wv_ref, bv_ref,
                a2_ref, bd1_ref, b2_ref,
                mu_ref, lv_ref, t2_ref, ci_ref):
    s = acc_ref[0, :_N, :] + acc_ref[1, :_N, :]
    inv = 1.0 / jnp.maximum(s[:, _BIG:_BIG + 1], 1.0)
    h = s[:, :_BIG] * inv
    mu = jnp.dot(h, wmu_ref[...], preferred_element_type=_f32) + bmu_ref[...]
    lv = jnp.dot(h, wv_ref[...], preferred_element_type=_f32) + bv_ref[...]
    z = mu + eps_ref[...] * jnp.exp(0.5 * lv)
    mu_ref[...] = mu
    lv_ref[...] = lv
    p2 = jnp.dot(z, a2_ref[...], preferred_element_type=_f32) + bd1_ref[...]
    q2 = jnp.dot(z, b2_ref[...], preferred_element_type=_f32)
    t2_ref[...] = jnp.concatenate([p2, q2], axis=1)
    ci_ref[...] = jnp.broadcast_to(inv, (_N, 8))


def _out_body(acc_ref, ci_ref, o_ref):
    s = acc_ref[0, :_N, :] + acc_ref[1, :_N, :]
    o_ref[...] = s * ci_ref[:, 0:1]


def _sc_gather(table, idx_d, idx_s):
    mesh = plsc.VectorSubcoreMesh(core_axis_name="c", subcore_axis_name="s",
                                  num_cores=_NC, num_subcores=_NS)
    osh = jax.ShapeDtypeStruct((_NW, _KP, _CH, _D), _f32)

    @functools.partial(
        pl.kernel,
        out_type=(osh, osh),
        mesh=mesh,
        scratch_types=[
            pltpu.VMEM((_KP, _CH), jnp.int32),
            pltpu.VMEM((_KP, _CH), jnp.int32),
            pltpu.VMEM((_CH, _D), _f32),
            pltpu.VMEM((_CH, _D), _f32),
            pltpu.SemaphoreType.DMA,
        ],
    )
    def k(tab, ip, iq, od, os_, ipv, iqv, bd, bs, sem):
        cid = lax.axis_index("c")
        sid = lax.axis_index("s")
        wid = sid * _NC + cid
        pltpu.sync_copy(ip.at[wid], ipv)
        pltpu.sync_copy(iq.at[wid], iqv)

        @pl.loop(0, _KP)
        def _(j):
            pltpu.async_copy(tab.at[ipv.at[j]], bd, sem).wait()
            pltpu.sync_copy(bd, od.at[wid, j])
            pltpu.async_copy(tab.at[iqv.at[j]], bs, sem).wait()
            pltpu.sync_copy(bs, os_.at[wid, j])

    return k(table, idx_d, idx_s)


def _sc_scatter(msgs, idx, zeros):
    mesh = plsc.VectorSubcoreMesh(core_axis_name="c", subcore_axis_name="s",
                                  num_cores=_NC, num_subcores=_NS)

    @functools.partial(
        pl.kernel,
        out_type=jax.ShapeDtypeStruct((_NC, _NACC, _D), _f32),
        mesh=mesh,
        scratch_types=[
            pltpu.VMEM((_KP, _CH), jnp.int32),
            pltpu.VMEM((_CH, _D), _f32),
            pltpu.VMEM_SHARED((_NACC, _D), _f32),
        ],
    )
    def k(m, ix, z, out, ixv, buf, acc):
        cid = lax.axis_index("c")
        sid = lax.axis_index("s")
        wid = sid * _NC + cid
        r0 = sid * _RPT
        pltpu.sync_copy(z.at[pl.ds(r0, _RPT)], acc.at[pl.ds(r0, _RPT)])
        plsc.subcore_barrier()
        pltpu.sync_copy(ix.at[wid], ixv)

        @pl.loop(0, _KP)
        def _(j):
            pltpu.sync_copy(m.at[wid, j], buf)
            pltpu.sync_copy(buf, acc.at[ixv.at[j]], add=True)

        plsc.subcore_barrier()
        pltpu.sync_copy(acc.at[pl.ds(r0, _RPT)], out.at[cid, pl.ds(r0, _RPT)])

    return k(msgs, idx, zeros)


def kernel(x, edge_index, gamma, beta, We1, be1, We2, be2, Wmu, bmu,
           Wvar, bvar, Wd1, bd1, Wd2, bd2, Wd3, bd3):
    src = edge_index[0].astype(jnp.int32)
    dst = edge_index[1].astype(jnp.int32)
    pad = _EP - _E
    zpad = jnp.zeros((pad,), jnp.int32)
    g_dst = jnp.concatenate([dst, zpad]).reshape(_NW, _KP, _CH)
    g_src = jnp.concatenate([src, zpad]).reshape(_NW, _KP, _CH)
    s_dst = jnp.concatenate([dst, jnp.full((pad,), _N, jnp.int32)]).reshape(_NW, _KP, _CH)

    A1 = (We1[:, :_D] - We1[:, _D:]).T
    B1 = We1[:, _D:].T
    A2 = (Wd1[:, :_HID] - Wd1[:, _HID:]).T
    B2 = Wd1[:, _HID:].T

    t1 = pl.pallas_call(
        _node1_body,
        out_shape=jax.ShapeDtypeStruct((_N, _D), _f32),
    )(x, gamma.reshape(1, _D), beta.reshape(1, _D), A1, B1, be1.reshape(1, _BIG))

    gd1, gs1 = _sc_gather(t1, g_dst, g_src)

    m1 = pl.pallas_call(
        _edge1_body,
        grid=(_EP // _BE,),
        in_specs=[
            pl.BlockSpec((_BE, _D), lambda i: (i, 0)),
            pl.BlockSpec((_BE, _D), lambda i: (i, 0)),
            pl.BlockSpec((_BIG, _BIG), lambda i: (0, 0)),
            pl.BlockSpec((1, _BIG), lambda i: (0, 0)),
        ],
        out_specs=pl.BlockSpec((_BE, _D), lambda i: (i, 0)),
        out_shape=jax.ShapeDtypeStruct((_EP, _D), _f32),
        compiler_params=pltpu.CompilerParams(
            dimension_semantics=("arbitrary",)),
    )(gd1.reshape(_EP, _D), gs1.reshape(_EP, _D), We2.T, be2.reshape(1, _BIG))

    acc1 = _sc_scatter(m1.reshape(_NW, _KP, _CH, _D), s_dst,
                       jnp.zeros((_NACC, _D), _f32))

    eps = jax.random.normal(jax.random.key(42), (_N, _HID), _f32)
    mu, lv, t2, cinv = pl.pallas_call(
        _node2_body,
        out_shape=(jax.ShapeDtypeStruct((_N, _HID), _f32),
                   jax.ShapeDtypeStruct((_N, _HID), _f32),
                   jax.ShapeDtypeStruct((_N, _D), _f32),
                   jax.ShapeDtypeStruct((_N, 8), _f32)),
    )(acc1, eps, Wmu.T, bmu.reshape(1, _HID), Wvar.T, bvar.reshape(1, _HID),
      A2, bd1.reshape(1, _BIG), B2)

    gd2, gs2 = _sc_gather(t2, g_dst, g_src)

    m2 = pl.pallas_call(
        _edge2_body,
        grid=(_EP // _BE,),
        in_specs=[
            pl.BlockSpec((_BE, _D), lambda i: (i, 0)),
            pl.BlockSpec((_BE, _D), lambda i: (i, 0)),
            pl.BlockSpec((_BIG, _BIG), lambda i: (0, 0)),
            pl.BlockSpec((1, _BIG), lambda i: (0, 0)),
            pl.BlockSpec((_BIG, _D), lambda i: (0, 0)),
            pl.BlockSpec((1, _D), lambda i: (0, 0)),
        ],
        out_specs=pl.BlockSpec((_BE, _D), lambda i: (i, 0)),
        out_shape=jax.ShapeDtypeStruct((_EP, _D), _f32),
        compiler_params=pltpu.CompilerParams(
            dimension_semantics=("arbitrary",)),
    )(gd2.reshape(_EP, _D), gs2.reshape(_EP, _D), Wd2.T, bd2.reshape(1, _BIG),
      Wd3.T, bd3.reshape(1, _D))

    acc2 = _sc_scatter(m2.reshape(_NW, _KP, _CH, _D), s_dst,
                       jnp.zeros((_NACC, _D), _f32))

    out = pl.pallas_call(
        _out_body,
        out_shape=jax.ShapeDtypeStruct((_N, _D), _f32),
    )(acc2, cinv)

    return (out, mu, lv)


# speedup vs baseline: 1.7448x; 1.1125x over previous
import functools

import jax
import jax.numpy as jnp
from jax import lax
from jax.experimental import pallas as pl
from jax.experimental.pallas import tpu as pltpu
from jax.experimental.pallas import tpu_sc as plsc

_N = 10000
_E = 320000
_D = 128
_BIG = 64
_HID = 32
_NC = 2
_NS = 16
_NW = _NC * _NS
_CH = 128
_KP = 80
_EPT = _KP * _CH
_EP = _NW * _EPT
_NACC = 10112
_RPT = _NACC // _NS
_BE = 1024

_f32 = jnp.float32


def _node1_body(x_ref, g_ref, b_ref, a1_ref, b1_ref, be1_ref, t_ref):
    x = x_ref[...]
    mean = jnp.mean(x, axis=0, keepdims=True)
    xc = x - mean
    var = jnp.mean(xc * xc, axis=0, keepdims=True)
    xn = xc * lax.rsqrt(var + 1e-5) * g_ref[...] + b_ref[...]
    p = jnp.dot(xn, a1_ref[...], preferred_element_type=_f32) + be1_ref[...]
    q = jnp.dot(xn, b1_ref[...], preferred_element_type=_f32)
    t_ref[...] = jnp.concatenate([p, q], axis=1)


def _edge1_body(gd_ref, gs_ref, w2_ref, b2_ref, m_ref):
    pre = jnp.maximum(gd_ref[:, :_BIG] + gs_ref[:, _BIG:], 0.0)
    m = jnp.dot(pre, w2_ref[...], preferred_element_type=_f32) + b2_ref[...]
    m = jnp.maximum(m, 0.0)
    m_ref[...] = jnp.concatenate([m, jnp.ones((_BE, _BIG), _f32)], axis=1)


def _edge2_body(gd_ref, gs_ref, w2_ref, b2_ref, w3_ref, b3_ref, m_ref):
    pre = jnp.maximum(gd_ref[:, :_BIG] + gs_ref[:, _BIG:], 0.0)
    t = jnp.dot(pre, w2_ref[...], preferred_element_type=_f32) + b2_ref[...]
    t = jnp.maximum(t, 0.0)
    m_ref[...] = jnp.dot(t, w3_ref[...], preferred_element_type=_f32) + b3_ref[...]


def _node2_body(acc_ref, eps_ref, wmu_ref, bmu_ref, wv_ref, bv_ref,
                a2_ref, bd1_ref, b2_ref,
                mu_ref, lv_ref, t2_ref, ci_ref):
    s = acc_ref[0, :_N, :] + acc_ref[1, :_N, :]
    inv = 1.0 / jnp.maximum(s[:, _BIG:_BIG + 1], 1.0)
    h = s[:, :_BIG] * inv
    mu = jnp.dot(h, wmu_ref[...], preferred_element_type=_f32) + bmu_ref[...]
    lv = jnp.dot(h, wv_ref[...], preferred_element_type=_f32) + bv_ref[...]
    z = mu + eps_ref[...] * jnp.exp(0.5 * lv)
    mu_ref[...] = mu
    lv_ref[...] = lv
    p2 = jnp.dot(z, a2_ref[...], preferred_element_type=_f32) + bd1_ref[...]
    q2 = jnp.dot(z, b2_ref[...], preferred_element_type=_f32)
    t2_ref[...] = jnp.concatenate([p2, q2], axis=1)
    ci_ref[...] = jnp.broadcast_to(inv, (_N, 8))


def _out_body(acc_ref, ci_ref, o_ref):
    s = acc_ref[0, :_N, :] + acc_ref[1, :_N, :]
    o_ref[...] = s * ci_ref[:, 0:1]


def _sc_gather(table, idx_d, idx_s):
    mesh = plsc.VectorSubcoreMesh(core_axis_name="c", subcore_axis_name="s",
                                  num_cores=_NC, num_subcores=_NS)
    osh = jax.ShapeDtypeStruct((_NW, _KP, _CH, _D), _f32)

    @functools.partial(
        pl.kernel,
        out_type=(osh, osh),
        mesh=mesh,
        scratch_types=[
            pltpu.VMEM((_KP, _CH), jnp.int32),
            pltpu.VMEM((_KP, _CH), jnp.int32),
            pltpu.VMEM((4, _CH, _D), _f32),
            pltpu.SemaphoreType.DMA,
            pltpu.SemaphoreType.DMA,
        ],
    )
    def k(tab, ip, iq, od, os_, ipv, iqv, bufs, gsem, wsem):
        cid = lax.axis_index("c")
        sid = lax.axis_index("s")
        wid = sid * _NC + cid
        pltpu.sync_copy(ip.at[wid], ipv)
        pltpu.sync_copy(iq.at[wid], iqv)

        def run_pass(ixv, out):
            for b in range(2):
                pltpu.async_copy(tab.at[ixv.at[b]], bufs.at[b], gsem)

            @pl.loop(0, _KP // 4)
            def _(g):
                for b in range(4):
                    j = g * 4 + b
                    pltpu.make_async_copy(tab.at[ixv.at[j]], bufs.at[b], gsem).wait()
                    pltpu.async_copy(bufs.at[b], out.at[wid, j], wsem)

                    @pl.when(j >= 2)
                    def _():
                        pltpu.make_async_copy(bufs.at[b], out.at[wid, j], wsem).wait()

                    @pl.when(j + 2 < _KP)
                    def _():
                        pltpu.async_copy(tab.at[ixv.at[j + 2]], bufs.at[(b + 2) % 4], gsem)

            for b in range(2):
                pltpu.make_async_copy(bufs.at[b], out.at[wid, 0], wsem).wait()

        run_pass(ipv, od)
        run_pass(iqv, os_)

    return k(table, idx_d, idx_s)


def _sc_scatter(msgs, idx, zeros):
    mesh = plsc.VectorSubcoreMesh(core_axis_name="c", subcore_axis_name="s",
                                  num_cores=_NC, num_subcores=_NS)

    @functools.partial(
        pl.kernel,
        out_type=jax.ShapeDtypeStruct((_NC, _NACC, _D), _f32),
        mesh=mesh,
        scratch_types=[
            pltpu.VMEM((_KP, _CH), jnp.int32),
            pltpu.VMEM((2, _CH, _D), _f32),
            pltpu.VMEM_SHARED((_NACC, _D), _f32),
            pltpu.SemaphoreType.DMA,
            pltpu.SemaphoreType.DMA,
        ],
    )
    def k(m, ix, z, out, ixv, bufs, acc, lsem, ssem):
        cid = lax.axis_index("c")
        sid = lax.axis_index("s")
        wid = sid * _NC + cid
        r0 = sid * _RPT
        pltpu.sync_copy(z.at[pl.ds(r0, _RPT)], acc.at[pl.ds(r0, _RPT)])
        plsc.subcore_barrier()
        pltpu.sync_copy(ix.at[wid], ixv)

        pltpu.async_copy(m.at[wid, 0], bufs.at[0], lsem)

        @pl.loop(0, _KP // 2)
        def _(g):
            for b in range(2):
                j = g * 2 + b
                pltpu.make_async_copy(m.at[wid, j], bufs.at[b], lsem).wait()
                pltpu.async_copy(bufs.at[b], acc.at[ixv.at[j]], ssem, add=True)

                @pl.when(j >= 1)
                def _():
                    pltpu.make_async_copy(bufs.at[b], acc.at[ixv.at[j]], ssem).wait()

                @pl.when(j + 1 < _KP)
                def _():
                    pltpu.async_copy(m.at[wid, j + 1], bufs.at[(b + 1) % 2], lsem)

        pltpu.make_async_copy(bufs.at[0], acc.at[ixv.at[0]], ssem).wait()

        plsc.subcore_barrier()
        pltpu.sync_copy(acc.at[pl.ds(r0, _RPT)], out.at[cid, pl.ds(r0, _RPT)])

    return k(msgs, idx, zeros)


def kernel(x, edge_index, gamma, beta, We1, be1, We2, be2, Wmu, bmu,
           Wvar, bvar, Wd1, bd1, Wd2, bd2, Wd3, bd3):
    src = edge_index[0].astype(jnp.int32)
    dst = edge_index[1].astype(jnp.int32)
    pad = _EP - _E
    zpad = jnp.zeros((pad,), jnp.int32)
    g_dst = jnp.concatenate([dst, zpad]).reshape(_NW, _KP, _CH)
    g_src = jnp.concatenate([src, zpad]).reshape(_NW, _KP, _CH)
    s_dst = jnp.concatenate([dst, jnp.full((pad,), _N, jnp.int32)]).reshape(_NW, _KP, _CH)

    A1 = (We1[:, :_D] - We1[:, _D:]).T
    B1 = We1[:, _D:].T
    A2 = (Wd1[:, :_HID] - Wd1[:, _HID:]).T
    B2 = Wd1[:, _HID:].T

    t1 = pl.pallas_call(
        _node1_body,
        out_shape=jax.ShapeDtypeStruct((_N, _D), _f32),
    )(x, gamma.reshape(1, _D), beta.reshape(1, _D), A1, B1, be1.reshape(1, _BIG))

    gd1, gs1 = _sc_gather(t1, g_dst, g_src)

    m1 = pl.pallas_call(
        _edge1_body,
        grid=(_EP // _BE,),
        in_specs=[
            pl.BlockSpec((_BE, _D), lambda i: (i, 0)),
            pl.BlockSpec((_BE, _D), lambda i: (i, 0)),
            pl.BlockSpec((_BIG, _BIG), lambda i: (0, 0)),
            pl.BlockSpec((1, _BIG), lambda i: (0, 0)),
        ],
        out_specs=pl.BlockSpec((_BE, _D), lambda i: (i, 0)),
        out_shape=jax.ShapeDtypeStruct((_EP, _D), _f32),
        compiler_params=pltpu.CompilerParams(
            dimension_semantics=("arbitrary",)),
    )(gd1.reshape(_EP, _D), gs1.reshape(_EP, _D), We2.T, be2.reshape(1, _BIG))

    acc1 = _sc_scatter(m1.reshape(_NW, _KP, _CH, _D), s_dst,
                       jnp.zeros((_NACC, _D), _f32))

    eps = jax.random.normal(jax.random.key(42), (_N, _HID), _f32)
    mu, lv, t2, cinv = pl.pallas_call(
        _node2_body,
        out_shape=(jax.ShapeDtypeStruct((_N, _HID), _f32),
                   jax.ShapeDtypeStruct((_N, _HID), _f32),
                   jax.ShapeDtypeStruct((_N, _D), _f32),
                   jax.ShapeDtypeStruct((_N, 8), _f32)),
    )(acc1, eps, Wmu.T, bmu.reshape(1, _HID), Wvar.T, bvar.reshape(1, _HID),
      A2, bd1.reshape(1, _BIG), B2)

    gd2, gs2 = _sc_gather(t2, g_dst, g_src)

    m2 = pl.pallas_call(
        _edge2_body,
        grid=(_EP // _BE,),
        in_specs=[
            pl.BlockSpec((_BE, _D), lambda i: (i, 0)),
            pl.BlockSpec((_BE, _D), lambda i: (i, 0)),
            pl.BlockSpec((_BIG, _BIG), lambda i: (0, 0)),
            pl.BlockSpec((1, _BIG), lambda i: (0, 0)),
            pl.BlockSpec((_BIG, _D), lambda i: (0, 0)),
            pl.BlockSpec((1, _D), lambda i: (0, 0)),
        ],
        out_specs=pl.BlockSpec((_BE, _D), lambda i: (i, 0)),
        out_shape=jax.ShapeDtypeStruct((_EP, _D), _f32),
        compiler_params=pltpu.CompilerParams(
            dimension_semantics=("arbitrary",)),
    )(gd2.reshape(_EP, _D), gs2.reshape(_EP, _D), Wd2.T, bd2.reshape(1, _BIG),
      Wd3.T, bd3.reshape(1, _D))

    acc2 = _sc_scatter(m2.reshape(_NW, _KP, _CH, _D), s_dst,
                       jnp.zeros((_NACC, _D), _f32))

    out = pl.pallas_call(
        _out_body,
        out_shape=jax.ShapeDtypeStruct((_N, _D), _f32),
    )(acc2, cinv)

    return (out, mu, lv)


# speedup vs baseline: 4.2502x; 2.4360x over previous
import functools

import jax
import jax.numpy as jnp
from jax import lax
from jax.experimental import pallas as pl
from jax.experimental.pallas import tpu as pltpu
from jax.experimental.pallas import tpu_sc as plsc

_N = 10000
_E = 320000
_D = 128
_BIG = 64
_HID = 32
_NC = 2
_NS = 16
_NW = _NC * _NS
_CH = 128
_KP = 80
_EPT = _KP * _CH
_EP = _NW * _EPT
_NACC = 10112
_RPT = _NACC // _NS
_BE = 1024

_f32 = jnp.float32


def _node1_body(x_ref, g_ref, b_ref, a1_ref, b1_ref, be1_ref, t_ref):
    x = x_ref[...]
    mean = jnp.mean(x, axis=0, keepdims=True)
    xc = x - mean
    var = jnp.mean(xc * xc, axis=0, keepdims=True)
    xn = xc * lax.rsqrt(var + 1e-5) * g_ref[...] + b_ref[...]
    p = jnp.dot(xn, a1_ref[...], preferred_element_type=_f32) + be1_ref[...]
    q = jnp.dot(xn, b1_ref[...], preferred_element_type=_f32)
    t_ref[:_N, :] = jnp.concatenate([p, q], axis=1)


def _edge1_body(gd_ref, gs_ref, w2_ref, b2_ref, m_ref):
    pre = jnp.maximum(gd_ref[:, :_BIG] + gs_ref[:, _BIG:], 0.0)
    m = jnp.dot(pre, w2_ref[...], preferred_element_type=_f32) + b2_ref[...]
    m = jnp.maximum(m, 0.0)
    m_ref[...] = jnp.concatenate([m, jnp.ones((_BE, _BIG), _f32)], axis=1)


def _edge2_body(gd_ref, gs_ref, w2_ref, b2_ref, w3_ref, b3_ref, m_ref):
    pre = jnp.maximum(gd_ref[:, :_BIG] + gs_ref[:, _BIG:], 0.0)
    t = jnp.dot(pre, w2_ref[...], preferred_element_type=_f32) + b2_ref[...]
    t = jnp.maximum(t, 0.0)
    m_ref[...] = jnp.dot(t, w3_ref[...], preferred_element_type=_f32) + b3_ref[...]


def _node2_body(acc_ref, eps_ref, wmu_ref, bmu_ref, wv_ref, bv_ref,
                a2_ref, bd1_ref, b2_ref,
                mu_ref, lv_ref, t2_ref, ci_ref):
    s = acc_ref[0, :_N, :] + acc_ref[1, :_N, :]
    inv = 1.0 / jnp.maximum(s[:, _BIG:_BIG + 1], 1.0)
    h = s[:, :_BIG] * inv
    mu = jnp.dot(h, wmu_ref[...], preferred_element_type=_f32) + bmu_ref[...]
    lv = jnp.dot(h, wv_ref[...], preferred_element_type=_f32) + bv_ref[...]
    z = mu + eps_ref[...] * jnp.exp(0.5 * lv)
    mu_ref[...] = mu
    lv_ref[...] = lv
    p2 = jnp.dot(z, a2_ref[...], preferred_element_type=_f32) + bd1_ref[...]
    q2 = jnp.dot(z, b2_ref[...], preferred_element_type=_f32)
    t2_ref[:_N, :] = jnp.concatenate([p2, q2], axis=1)
    ci_ref[...] = jnp.broadcast_to(inv, (_N, 8))


def _out_body(acc_ref, ci_ref, o_ref):
    s = acc_ref[0, :_N, :] + acc_ref[1, :_N, :]
    o_ref[...] = s * ci_ref[:, 0:1]


def _sc_gather(table, idx_d, idx_s):
    mesh = plsc.VectorSubcoreMesh(core_axis_name="c", subcore_axis_name="s",
                                  num_cores=_NC, num_subcores=_NS)
    osh = jax.ShapeDtypeStruct((_NW, _KP, _CH, _D), _f32)

    @functools.partial(
        pl.kernel,
        out_type=(osh, osh),
        mesh=mesh,
        scratch_types=[
            pltpu.VMEM((_KP, _CH), jnp.int32),
            pltpu.VMEM((2, _CH, _D), _f32),
            pltpu.VMEM_SHARED((_NACC, _D), _f32),
            pltpu.SemaphoreType.DMA,
            pltpu.SemaphoreType.DMA,
        ],
    )
    def k(tab, ip, iq, od, os_, ixv, bufs, tabs, gsem, wsem):
        cid = lax.axis_index("c")
        sid = lax.axis_index("s")
        wid = sid * _NC + cid
        r0 = sid * _RPT
        pltpu.sync_copy(tab.at[pl.ds(r0, _RPT)], tabs.at[pl.ds(r0, _RPT)])
        plsc.subcore_barrier()

        def run_pass(ix, out):
            pltpu.sync_copy(ix.at[wid], ixv)
            pltpu.async_copy(tabs.at[ixv.at[0]], bufs.at[0], gsem)

            @pl.loop(0, _KP // 2)
            def _(g):
                for b in range(2):
                    j = g * 2 + b
                    pltpu.make_async_copy(tabs.at[ixv.at[j]], bufs.at[b], gsem).wait()
                    pltpu.async_copy(bufs.at[b], out.at[wid, j], wsem)

                    @pl.when(j >= 1)
                    def _():
                        pltpu.make_async_copy(bufs.at[b], out.at[wid, j], wsem).wait()

                    @pl.when(j + 1 < _KP)
                    def _():
                        pltpu.async_copy(tabs.at[ixv.at[j + 1]], bufs.at[(b + 1) % 2], gsem)

            pltpu.make_async_copy(bufs.at[0], out.at[wid, 0], wsem).wait()

        run_pass(ip, od)
        run_pass(iq, os_)

    return k(table, idx_d, idx_s)


def _sc_scatter(msgs, idx, zeros):
    mesh = plsc.VectorSubcoreMesh(core_axis_name="c", subcore_axis_name="s",
                                  num_cores=_NC, num_subcores=_NS)

    @functools.partial(
        pl.kernel,
        out_type=jax.ShapeDtypeStruct((_NC, _NACC, _D), _f32),
        mesh=mesh,
        scratch_types=[
            pltpu.VMEM((_KP, _CH), jnp.int32),
            pltpu.VMEM((2, _CH, _D), _f32),
            pltpu.VMEM_SHARED((_NACC, _D), _f32),
            pltpu.SemaphoreType.DMA,
            pltpu.SemaphoreType.DMA,
        ],
    )
    def k(m, ix, z, out, ixv, bufs, acc, lsem, ssem):
        cid = lax.axis_index("c")
        sid = lax.axis_index("s")
        wid = sid * _NC + cid
        r0 = sid * _RPT
        pltpu.sync_copy(z.at[pl.ds(r0, _RPT)], acc.at[pl.ds(r0, _RPT)])
        plsc.subcore_barrier()
        pltpu.sync_copy(ix.at[wid], ixv)

        pltpu.async_copy(m.at[wid, 0], bufs.at[0], lsem)

        @pl.loop(0, _KP // 2)
        def _(g):
            for b in range(2):
                j = g * 2 + b
                pltpu.make_async_copy(m.at[wid, j], bufs.at[b], lsem).wait()
                pltpu.async_copy(bufs.at[b], acc.at[ixv.at[j]], ssem, add=True)

                @pl.when(j >= 1)
                def _():
                    pltpu.make_async_copy(bufs.at[b], acc.at[ixv.at[j]], ssem).wait()

                @pl.when(j + 1 < _KP)
                def _():
                    pltpu.async_copy(m.at[wid, j + 1], bufs.at[(b + 1) % 2], lsem)

        pltpu.make_async_copy(bufs.at[0], acc.at[ixv.at[0]], ssem).wait()

        plsc.subcore_barrier()
        pltpu.sync_copy(acc.at[pl.ds(r0, _RPT)], out.at[cid, pl.ds(r0, _RPT)])

    return k(msgs, idx, zeros)


def kernel(x, edge_index, gamma, beta, We1, be1, We2, be2, Wmu, bmu,
           Wvar, bvar, Wd1, bd1, Wd2, bd2, Wd3, bd3):
    src = edge_index[0].astype(jnp.int32)
    dst = edge_index[1].astype(jnp.int32)
    pad = _EP - _E
    zpad = jnp.zeros((pad,), jnp.int32)
    g_dst = jnp.concatenate([dst, zpad]).reshape(_NW, _KP, _CH)
    g_src = jnp.concatenate([src, zpad]).reshape(_NW, _KP, _CH)
    s_dst = jnp.concatenate([dst, jnp.full((pad,), _N, jnp.int32)]).reshape(_NW, _KP, _CH)

    A1 = (We1[:, :_D] - We1[:, _D:]).T
    B1 = We1[:, _D:].T
    A2 = (Wd1[:, :_HID] - Wd1[:, _HID:]).T
    B2 = Wd1[:, _HID:].T

    t1 = pl.pallas_call(
        _node1_body,
        out_shape=jax.ShapeDtypeStruct((_NACC, _D), _f32),
    )(x, gamma.reshape(1, _D), beta.reshape(1, _D), A1, B1, be1.reshape(1, _BIG))

    gd1, gs1 = _sc_gather(t1, g_dst, g_src)

    m1 = pl.pallas_call(
        _edge1_body,
        grid=(_EP // _BE,),
        in_specs=[
            pl.BlockSpec((_BE, _D), lambda i: (i, 0)),
            pl.BlockSpec((_BE, _D), lambda i: (i, 0)),
            pl.BlockSpec((_BIG, _BIG), lambda i: (0, 0)),
            pl.BlockSpec((1, _BIG), lambda i: (0, 0)),
        ],
        out_specs=pl.BlockSpec((_BE, _D), lambda i: (i, 0)),
        out_shape=jax.ShapeDtypeStruct((_EP, _D), _f32),
        compiler_params=pltpu.CompilerParams(
            dimension_semantics=("arbitrary",)),
    )(gd1.reshape(_EP, _D), gs1.reshape(_EP, _D), We2.T, be2.reshape(1, _BIG))

    acc1 = _sc_scatter(m1.reshape(_NW, _KP, _CH, _D), s_dst,
                       jnp.zeros((_NACC, _D), _f32))

    eps = jax.random.normal(jax.random.key(42), (_N, _HID), _f32)
    mu, lv, t2, cinv = pl.pallas_call(
        _node2_body,
        out_shape=(jax.ShapeDtypeStruct((_N, _HID), _f32),
                   jax.ShapeDtypeStruct((_N, _HID), _f32),
                   jax.ShapeDtypeStruct((_NACC, _D), _f32),
                   jax.ShapeDtypeStruct((_N, 8), _f32)),
    )(acc1, eps, Wmu.T, bmu.reshape(1, _HID), Wvar.T, bvar.reshape(1, _HID),
      A2, bd1.reshape(1, _BIG), B2)

    gd2, gs2 = _sc_gather(t2, g_dst, g_src)

    m2 = pl.pallas_call(
        _edge2_body,
        grid=(_EP // _BE,),
        in_specs=[
            pl.BlockSpec((_BE, _D), lambda i: (i, 0)),
            pl.BlockSpec((_BE, _D), lambda i: (i, 0)),
            pl.BlockSpec((_BIG, _BIG), lambda i: (0, 0)),
            pl.BlockSpec((1, _BIG), lambda i: (0, 0)),
            pl.BlockSpec((_BIG, _D), lambda i: (0, 0)),
            pl.BlockSpec((1, _D), lambda i: (0, 0)),
        ],
        out_specs=pl.BlockSpec((_BE, _D), lambda i: (i, 0)),
        out_shape=jax.ShapeDtypeStruct((_EP, _D), _f32),
        compiler_params=pltpu.CompilerParams(
            dimension_semantics=("arbitrary",)),
    )(gd2.reshape(_EP, _D), gs2.reshape(_EP, _D), Wd2.T, bd2.reshape(1, _BIG),
      Wd3.T, bd3.reshape(1, _D))

    acc2 = _sc_scatter(m2.reshape(_NW, _KP, _CH, _D), s_dst,
                       jnp.zeros((_NACC, _D), _f32))

    out = pl.pallas_call(
        _out_body,
        out_shape=jax.ShapeDtypeStruct((_N, _D), _f32),
    )(acc2, cinv)

    return (out, mu, lv)


# speedup vs baseline: 4.8177x; 1.1335x over previous
import functools

import jax
import jax.numpy as jnp
from jax import lax
from jax.experimental import pallas as pl
from jax.experimental.pallas import tpu as pltpu
from jax.experimental.pallas import tpu_sc as plsc

_N = 10000
_E = 320000
_D = 128
_BIG = 64
_HID = 32
_NC = 2
_NS = 16
_NW = _NC * _NS
_CH = 128
_KP = 80
_EPT = _KP * _CH
_EP = _NW * _EPT
_NACC = 10240
_RPT = _NACC // _NS
_BE = 1024

_f32 = jnp.float32
_bf16 = jnp.bfloat16


def _node1_body(x_ref, g_ref, b_ref, a1_ref, b1_ref, be1_ref, t_ref):
    x = x_ref[...]
    mean = jnp.mean(x, axis=0, keepdims=True)
    xc = x - mean
    var = jnp.mean(xc * xc, axis=0, keepdims=True)
    xn = xc * lax.rsqrt(var + 1e-5) * g_ref[...] + b_ref[...]
    p = jnp.dot(xn, a1_ref[...], preferred_element_type=_f32) + be1_ref[...]
    q = jnp.dot(xn, b1_ref[...], preferred_element_type=_f32)
    t_ref[:_N, :] = jnp.concatenate([p, q], axis=1)


def _edge1_body(gd_ref, gs_ref, w2_ref, b2_ref, m_ref):
    pre = jnp.maximum(gd_ref[:, :_BIG] + gs_ref[:, _BIG:], 0.0)
    m = jnp.dot(pre, w2_ref[...], preferred_element_type=_f32) + b2_ref[...]
    m = jnp.maximum(m, 0.0)
    m_ref[...] = jnp.concatenate([m, jnp.ones((_BE, _BIG), _f32)], axis=1)


def _edge2_body(gd_ref, gs_ref, w2_ref, b2_ref, w3_ref, b3_ref, m_ref):
    pre = jnp.maximum(gd_ref[:, :_BIG] + gs_ref[:, _BIG:], 0.0)
    t = jnp.dot(pre, w2_ref[...], preferred_element_type=_f32) + b2_ref[...]
    t = jnp.maximum(t, 0.0)
    m_ref[...] = jnp.dot(t, w3_ref[...], preferred_element_type=_f32) + b3_ref[...]


def _node2_body(acca_ref, accb_ref, eps_ref, wmu_ref, bmu_ref, wv_ref, bv_ref,
                a2_ref, bd1_ref, b2_ref,
                mu_ref, lv_ref, t2_ref, ci_ref):
    s = (acca_ref[0, :_N, :] + acca_ref[1, :_N, :]
         + accb_ref[0, :_N, :] + accb_ref[1, :_N, :])
    inv = 1.0 / jnp.maximum(s[:, _BIG:_BIG + 1], 1.0)
    h = s[:, :_BIG] * inv
    mu = jnp.dot(h, wmu_ref[...], preferred_element_type=_f32) + bmu_ref[...]
    lv = jnp.dot(h, wv_ref[...], preferred_element_type=_f32) + bv_ref[...]
    z = mu + eps_ref[...] * jnp.exp(0.5 * lv)
    mu_ref[...] = mu
    lv_ref[...] = lv
    p2 = jnp.dot(z, a2_ref[...], preferred_element_type=_f32) + bd1_ref[...]
    q2 = jnp.dot(z, b2_ref[...], preferred_element_type=_f32)
    t2_ref[:_N, :] = jnp.concatenate([p2, q2], axis=1)
    ci_ref[...] = jnp.broadcast_to(inv, (_N, 8))


def _out_body(acca_ref, accb_ref, ci_ref, o_ref):
    s = (acca_ref[0, :_N, :] + acca_ref[1, :_N, :]
         + accb_ref[0, :_N, :] + accb_ref[1, :_N, :])
    o_ref[...] = s * ci_ref[:, 0:1]


def _sc_gather(table, idx_d, idx_s, kp):
    mesh = plsc.VectorSubcoreMesh(core_axis_name="c", subcore_axis_name="s",
                                  num_cores=_NC, num_subcores=_NS)
    osh = jax.ShapeDtypeStruct((_NW, kp, _CH, _D), _f32)

    @functools.partial(
        pl.kernel,
        out_type=(osh, osh),
        mesh=mesh,
        scratch_types=[
            pltpu.VMEM((kp, _CH), jnp.int32),
            pltpu.VMEM((2, _CH, _D), _f32),
            pltpu.VMEM_SHARED((_NACC, _D), _f32),
            pltpu.SemaphoreType.DMA,
            pltpu.SemaphoreType.DMA,
        ],
    )
    def k(tab, ip, iq, od, os_, ixv, bufs, tabs, gsem, wsem):
        cid = lax.axis_index("c")
        sid = lax.axis_index("s")
        wid = sid * _NC + cid
        r0 = sid * _RPT
        pltpu.sync_copy(tab.at[pl.ds(r0, _RPT)], tabs.at[pl.ds(r0, _RPT)])
        plsc.subcore_barrier()

        def run_pass(ix, out):
            pltpu.sync_copy(ix.at[wid], ixv)
            pltpu.async_copy(tabs.at[ixv.at[0]], bufs.at[0], gsem)

            @pl.loop(0, kp // 2)
            def _(g):
                for b in range(2):
                    j = g * 2 + b
                    pltpu.make_async_copy(tabs.at[ixv.at[j]], bufs.at[b], gsem).wait()
                    pltpu.async_copy(bufs.at[b], out.at[wid, j], wsem)

                    @pl.when(j >= 1)
                    def _():
                        pltpu.make_async_copy(bufs.at[b], out.at[wid, j], wsem).wait()

                    @pl.when(j + 1 < kp)
                    def _():
                        pltpu.async_copy(tabs.at[ixv.at[j + 1]], bufs.at[(b + 1) % 2], gsem)

            pltpu.make_async_copy(bufs.at[0], out.at[wid, 0], wsem).wait()

        run_pass(ip, od)
        run_pass(iq, os_)

    return k(table, idx_d, idx_s)


def _sc_scatter(msgs, idx, zeros, kp):
    mesh = plsc.VectorSubcoreMesh(core_axis_name="c", subcore_axis_name="s",
                                  num_cores=_NC, num_subcores=_NS)

    @functools.partial(
        pl.kernel,
        out_type=jax.ShapeDtypeStruct((_NC, _NACC, _D), _f32),
        mesh=mesh,
        scratch_types=[
            pltpu.VMEM((kp, _CH), jnp.int32),
            pltpu.VMEM((2, _CH, _D), _f32),
            pltpu.VMEM_SHARED((_NACC, _D), _f32),
            pltpu.SemaphoreType.DMA,
            pltpu.SemaphoreType.DMA,
        ],
    )
    def k(m, ix, z, out, ixv, bufs, acc, lsem, ssem):
        cid = lax.axis_index("c")
        sid = lax.axis_index("s")
        wid = sid * _NC + cid
        r0 = sid * _RPT
        pltpu.sync_copy(z.at[pl.ds(r0, _RPT)], acc.at[pl.ds(r0, _RPT)])
        plsc.subcore_barrier()
        pltpu.sync_copy(ix.at[wid], ixv)

        pltpu.async_copy(m.at[wid, 0], bufs.at[0], lsem)

        @pl.loop(0, kp // 2)
        def _(g):
            for b in range(2):
                j = g * 2 + b
                pltpu.make_async_copy(m.at[wid, j], bufs.at[b], lsem).wait()
                pltpu.async_copy(bufs.at[b], acc.at[ixv.at[j]], ssem, add=True)

                @pl.when(j >= 1)
                def _():
                    pltpu.make_async_copy(bufs.at[b], acc.at[ixv.at[j]], ssem).wait()

                @pl.when(j + 1 < kp)
                def _():
                    pltpu.async_copy(m.at[wid, j + 1], bufs.at[(b + 1) % 2], lsem)

        pltpu.make_async_copy(bufs.at[0], acc.at[ixv.at[0]], ssem).wait()

        plsc.subcore_barrier()
        pltpu.sync_copy(acc.at[pl.ds(r0, _RPT)], out.at[cid, pl.ds(r0, _RPT)])

    return k(msgs, idx, zeros)


_KPH = _KP // 2
_EPH = _EP // 2


def _edge_mlp1(gd, gs, w2t, b2r):
    return pl.pallas_call(
        _edge1_body,
        grid=(_EPH // _BE,),
        in_specs=[
            pl.BlockSpec((_BE, _D), lambda i: (i, 0)),
            pl.BlockSpec((_BE, _D), lambda i: (i, 0)),
            pl.BlockSpec((_BIG, _BIG), lambda i: (0, 0)),
            pl.BlockSpec((1, _BIG), lambda i: (0, 0)),
        ],
        out_specs=pl.BlockSpec((_BE, _D), lambda i: (i, 0)),
        out_shape=jax.ShapeDtypeStruct((_EPH, _D), _f32),
        compiler_params=pltpu.CompilerParams(
            dimension_semantics=("arbitrary",)),
    )(gd.reshape(_EPH, _D), gs.reshape(_EPH, _D), w2t, b2r)


def _edge_mlp2(gd, gs, w2t, b2r, w3t, b3r):
    return pl.pallas_call(
        _edge2_body,
        grid=(_EPH // _BE,),
        in_specs=[
            pl.BlockSpec((_BE, _D), lambda i: (i, 0)),
            pl.BlockSpec((_BE, _D), lambda i: (i, 0)),
            pl.BlockSpec((_BIG, _BIG), lambda i: (0, 0)),
            pl.BlockSpec((1, _BIG), lambda i: (0, 0)),
            pl.BlockSpec((_BIG, _D), lambda i: (0, 0)),
            pl.BlockSpec((1, _D), lambda i: (0, 0)),
        ],
        out_specs=pl.BlockSpec((_BE, _D), lambda i: (i, 0)),
        out_shape=jax.ShapeDtypeStruct((_EPH, _D), _f32),
        compiler_params=pltpu.CompilerParams(
            dimension_semantics=("arbitrary",)),
    )(gd.reshape(_EPH, _D), gs.reshape(_EPH, _D), w2t, b2r, w3t, b3r)


def kernel(x, edge_index, gamma, beta, We1, be1, We2, be2, Wmu, bmu,
           Wvar, bvar, Wd1, bd1, Wd2, bd2, Wd3, bd3):
    src = edge_index[0].astype(jnp.int32)
    dst = edge_index[1].astype(jnp.int32)
    pad = _EP - _E
    zpad = jnp.zeros((pad,), jnp.int32)
    g_dst = jnp.concatenate([dst, zpad]).reshape(_NW, _KP, _CH)
    g_src = jnp.concatenate([src, zpad]).reshape(_NW, _KP, _CH)
    s_dst = jnp.concatenate([dst, jnp.full((pad,), _N, jnp.int32)]).reshape(_NW, _KP, _CH)
    gda, gdb = g_dst[:, :_KPH], g_dst[:, _KPH:]
    gsa, gsb = g_src[:, :_KPH], g_src[:, _KPH:]
    sda, sdb = s_dst[:, :_KPH], s_dst[:, _KPH:]
    zacc = jnp.zeros((_NACC, _D), _f32)

    A1 = (We1[:, :_D] - We1[:, _D:]).T
    B1 = We1[:, _D:].T
    A2 = (Wd1[:, :_HID] - Wd1[:, _HID:]).T
    B2 = Wd1[:, _HID:].T

    t1 = pl.pallas_call(
        _node1_body,
        out_shape=jax.ShapeDtypeStruct((_NACC, _D), _f32),
    )(x, gamma.reshape(1, _D), beta.reshape(1, _D), A1, B1, be1.reshape(1, _BIG))

    gd1a, gs1a = _sc_gather(t1, gda, gsa, _KPH)
    gd1b, gs1b = _sc_gather(t1, gdb, gsb, _KPH)
    m1a = _edge_mlp1(gd1a, gs1a, We2.T, be2.reshape(1, _BIG))
    acc1a = _sc_scatter(m1a.reshape(_NW, _KPH, _CH, _D), sda, zacc, _KPH)
    m1b = _edge_mlp1(gd1b, gs1b, We2.T, be2.reshape(1, _BIG))
    acc1b = _sc_scatter(m1b.reshape(_NW, _KPH, _CH, _D), sdb, zacc, _KPH)

    eps = jax.random.normal(jax.random.key(42), (_N, _HID), _f32)
    mu, lv, t2, cinv = pl.pallas_call(
        _node2_body,
        out_shape=(jax.ShapeDtypeStruct((_N, _HID), _f32),
                   jax.ShapeDtypeStruct((_N, _HID), _f32),
                   jax.ShapeDtypeStruct((_NACC, _D), _f32),
                   jax.ShapeDtypeStruct((_N, 8), _f32)),
    )(acc1a, acc1b, eps, Wmu.T, bmu.reshape(1, _HID), Wvar.T, bvar.reshape(1, _HID),
      A2, bd1.reshape(1, _BIG), B2)

    gd2a, gs2a = _sc_gather(t2, gda, gsa, _KPH)
    gd2b, gs2b = _sc_gather(t2, gdb, gsb, _KPH)
    m2a = _edge_mlp2(gd2a, gs2a, Wd2.T, bd2.reshape(1, _BIG), Wd3.T, bd3.reshape(1, _D))
    acc2a = _sc_scatter(m2a.reshape(_NW, _KPH, _CH, _D), sda, zacc, _KPH)
    m2b = _edge_mlp2(gd2b, gs2b, Wd2.T, bd2.reshape(1, _BIG), Wd3.T, bd3.reshape(1, _D))
    acc2b = _sc_scatter(m2b.reshape(_NW, _KPH, _CH, _D), sdb, zacc, _KPH)

    out = pl.pallas_call(
        _out_body,
        out_shape=jax.ShapeDtypeStruct((_N, _D), _f32),
    )(acc2a, acc2b, cinv)

    return (out, mu, lv)


# speedup vs baseline: 5.4990x; 1.1414x over previous
import functools

import jax
import jax.numpy as jnp
from jax import lax
from jax.experimental import pallas as pl
from jax.experimental.pallas import tpu as pltpu
from jax.experimental.pallas import tpu_sc as plsc

_N = 10000
_E = 320000
_D = 128
_BIG = 64
_HID = 32
_NC = 2
_NS = 16
_NW = _NC * _NS
_CH = 128
_KP = 80
_EPT = _KP * _CH
_EP = _NW * _EPT
_NACC = 10240
_RPT = _NACC // _NS
_BE = 2048

_f32 = jnp.float32
_bf16 = jnp.bfloat16


def _node1_body(x_ref, g_ref, b_ref, a1_ref, b1_ref, be1_ref, t_ref):
    x = x_ref[...]
    mean = jnp.mean(x, axis=0, keepdims=True)
    xc = x - mean
    var = jnp.mean(xc * xc, axis=0, keepdims=True)
    xn = xc * lax.rsqrt(var + 1e-5) * g_ref[...] + b_ref[...]
    p = jnp.dot(xn, a1_ref[...], preferred_element_type=_f32) + be1_ref[...]
    q = jnp.dot(xn, b1_ref[...], preferred_element_type=_f32)
    t_ref[:_N, :] = jnp.concatenate([p, q], axis=1)


def _edge1_body(gd_ref, gs_ref, w2_ref, b2_ref, m_ref):
    pre = jnp.maximum(gd_ref[:, :_BIG] + gs_ref[:, _BIG:], 0.0)
    m = jnp.dot(pre, w2_ref[...], preferred_element_type=_f32) + b2_ref[...]
    m = jnp.maximum(m, 0.0)
    m_ref[...] = jnp.concatenate([m, jnp.ones((_BE, _BIG), _f32)], axis=1)


def _edge2_body(gd_ref, gs_ref, w2_ref, b2_ref, w3_ref, b3_ref, m_ref):
    pre = jnp.maximum(gd_ref[:, :_BIG] + gs_ref[:, _BIG:], 0.0)
    t = jnp.dot(pre, w2_ref[...], preferred_element_type=_f32) + b2_ref[...]
    t = jnp.maximum(t, 0.0)
    m_ref[...] = jnp.dot(t, w3_ref[...], preferred_element_type=_f32) + b3_ref[...]


def _node2_body(acc_ref, eps_ref, wmu_ref, bmu_ref, wv_ref, bv_ref,
                a2_ref, bd1_ref, b2_ref,
                mu_ref, lv_ref, t2_ref, ci_ref):
    s = acc_ref[0, :_N, :] + acc_ref[1, :_N, :]
    inv = 1.0 / jnp.maximum(s[:, _BIG:_BIG + 1], 1.0)
    h = s[:, :_BIG] * inv
    mu = jnp.dot(h, wmu_ref[...], preferred_element_type=_f32) + bmu_ref[...]
    lv = jnp.dot(h, wv_ref[...], preferred_element_type=_f32) + bv_ref[...]
    z = mu + eps_ref[...] * jnp.exp(0.5 * lv)
    mu_ref[...] = mu
    lv_ref[...] = lv
    p2 = jnp.dot(z, a2_ref[...], preferred_element_type=_f32) + bd1_ref[...]
    q2 = jnp.dot(z, b2_ref[...], preferred_element_type=_f32)
    t2_ref[:_N, :] = jnp.concatenate([p2, q2], axis=1)
    ci_ref[...] = jnp.broadcast_to(inv, (_N, 8))


def _out_body(acc_ref, ci_ref, o_ref):
    s = acc_ref[0, :_N, :] + acc_ref[1, :_N, :]
    o_ref[...] = s * ci_ref[:, 0:1]


def _sc_gather(table, idx_d, idx_s, kp):
    mesh = plsc.VectorSubcoreMesh(core_axis_name="c", subcore_axis_name="s",
                                  num_cores=_NC, num_subcores=_NS)
    osh = jax.ShapeDtypeStruct((_NW, kp, _CH, _D), _f32)

    @functools.partial(
        pl.kernel,
        out_type=(osh, osh),
        mesh=mesh,
        scratch_types=[
            pltpu.VMEM((kp, _CH), jnp.int32),
            pltpu.VMEM((2, _CH, _D), _f32),
            pltpu.VMEM_SHARED((_NACC, _D), _f32),
            pltpu.SemaphoreType.DMA,
            pltpu.SemaphoreType.DMA,
        ],
    )
    def k(tab, ip, iq, od, os_, ixv, bufs, tabs, gsem, wsem):
        cid = lax.axis_index("c")
        sid = lax.axis_index("s")
        wid = sid * _NC + cid
        r0 = sid * _RPT
        pltpu.sync_copy(tab.at[pl.ds(r0, _RPT)], tabs.at[pl.ds(r0, _RPT)])
        plsc.subcore_barrier()

        def run_pass(ix, out):
            pltpu.sync_copy(ix.at[wid], ixv)
            pltpu.async_copy(tabs.at[ixv.at[0]], bufs.at[0], gsem)

            @pl.loop(0, kp // 2)
            def _(g):
                for b in range(2):
                    j = g * 2 + b
                    pltpu.make_async_copy(tabs.at[ixv.at[j]], bufs.at[b], gsem).wait()
                    pltpu.async_copy(bufs.at[b], out.at[wid, j], wsem)

                    @pl.when(j >= 1)
                    def _():
                        pltpu.make_async_copy(bufs.at[b], out.at[wid, j], wsem).wait()

                    @pl.when(j + 1 < kp)
                    def _():
                        pltpu.async_copy(tabs.at[ixv.at[j + 1]], bufs.at[(b + 1) % 2], gsem)

            pltpu.make_async_copy(bufs.at[0], out.at[wid, 0], wsem).wait()

        run_pass(ip, od)
        run_pass(iq, os_)

    return k(table, idx_d, idx_s)


def _sc_scatter(msgs, idx, init, kp):
    mesh = plsc.VectorSubcoreMesh(core_axis_name="c", subcore_axis_name="s",
                                  num_cores=_NC, num_subcores=_NS)

    @functools.partial(
        pl.kernel,
        out_type=jax.ShapeDtypeStruct((_NC, _NACC, _D), _f32),
        mesh=mesh,
        scratch_types=[
            pltpu.VMEM((kp, _CH), jnp.int32),
            pltpu.VMEM((2, _CH, _D), _f32),
            pltpu.VMEM_SHARED((_NACC, _D), _f32),
            pltpu.SemaphoreType.DMA,
            pltpu.SemaphoreType.DMA,
        ],
    )
    def k(m, ix, z, out, ixv, bufs, acc, lsem, ssem):
        cid = lax.axis_index("c")
        sid = lax.axis_index("s")
        wid = sid * _NC + cid
        r0 = sid * _RPT
        pltpu.sync_copy(z.at[cid, pl.ds(r0, _RPT)], acc.at[pl.ds(r0, _RPT)])
        plsc.subcore_barrier()
        pltpu.sync_copy(ix.at[wid], ixv)

        pltpu.async_copy(m.at[wid, 0], bufs.at[0], lsem)

        @pl.loop(0, kp // 2)
        def _(g):
            for b in range(2):
                j = g * 2 + b
                pltpu.make_async_copy(m.at[wid, j], bufs.at[b], lsem).wait()
                pltpu.async_copy(bufs.at[b], acc.at[ixv.at[j]], ssem, add=True)

                @pl.when(j >= 1)
                def _():
                    pltpu.make_async_copy(bufs.at[b], acc.at[ixv.at[j]], ssem).wait()

                @pl.when(j + 1 < kp)
                def _():
                    pltpu.async_copy(m.at[wid, j + 1], bufs.at[(b + 1) % 2], lsem)

        pltpu.make_async_copy(bufs.at[0], acc.at[ixv.at[0]], ssem).wait()

        plsc.subcore_barrier()
        pltpu.sync_copy(acc.at[pl.ds(r0, _RPT)], out.at[cid, pl.ds(r0, _RPT)])

    return k(msgs, idx, init)


_NSL = 4
_KPQ = _KP // _NSL
_EPQ = _EP // _NSL


def _edge_mlp1(gd, gs, w2t, b2r):
    return pl.pallas_call(
        _edge1_body,
        grid=(_EPQ // _BE,),
        in_specs=[
            pl.BlockSpec((_BE, _D), lambda i: (i, 0)),
            pl.BlockSpec((_BE, _D), lambda i: (i, 0)),
            pl.BlockSpec((_BIG, _BIG), lambda i: (0, 0)),
            pl.BlockSpec((1, _BIG), lambda i: (0, 0)),
        ],
        out_specs=pl.BlockSpec((_BE, _D), lambda i: (i, 0)),
        out_shape=jax.ShapeDtypeStruct((_EPQ, _D), _f32),
        compiler_params=pltpu.CompilerParams(
            dimension_semantics=("arbitrary",)),
    )(gd.reshape(_EPQ, _D), gs.reshape(_EPQ, _D), w2t, b2r)


def _edge_mlp2(gd, gs, w2t, b2r, w3t, b3r):
    return pl.pallas_call(
        _edge2_body,
        grid=(_EPQ // _BE,),
        in_specs=[
            pl.BlockSpec((_BE, _D), lambda i: (i, 0)),
            pl.BlockSpec((_BE, _D), lambda i: (i, 0)),
            pl.BlockSpec((_BIG, _BIG), lambda i: (0, 0)),
            pl.BlockSpec((1, _BIG), lambda i: (0, 0)),
            pl.BlockSpec((_BIG, _D), lambda i: (0, 0)),
            pl.BlockSpec((1, _D), lambda i: (0, 0)),
        ],
        out_specs=pl.BlockSpec((_BE, _D), lambda i: (i, 0)),
        out_shape=jax.ShapeDtypeStruct((_EPQ, _D), _f32),
        compiler_params=pltpu.CompilerParams(
            dimension_semantics=("arbitrary",)),
    )(gd.reshape(_EPQ, _D), gs.reshape(_EPQ, _D), w2t, b2r, w3t, b3r)


def kernel(x, edge_index, gamma, beta, We1, be1, We2, be2, Wmu, bmu,
           Wvar, bvar, Wd1, bd1, Wd2, bd2, Wd3, bd3):
    src = edge_index[0].astype(jnp.int32)
    dst = edge_index[1].astype(jnp.int32)
    pad = _EP - _E
    zpad = jnp.zeros((pad,), jnp.int32)
    g_dst = jnp.concatenate([dst, zpad]).reshape(_NW, _KP, _CH)
    g_src = jnp.concatenate([src, zpad]).reshape(_NW, _KP, _CH)
    s_dst = jnp.concatenate([dst, jnp.full((pad,), _N, jnp.int32)]).reshape(_NW, _KP, _CH)
    gds = [g_dst[:, i * _KPQ:(i + 1) * _KPQ] for i in range(_NSL)]
    gss = [g_src[:, i * _KPQ:(i + 1) * _KPQ] for i in range(_NSL)]
    sds = [s_dst[:, i * _KPQ:(i + 1) * _KPQ] for i in range(_NSL)]
    zacc = jnp.zeros((_NC, _NACC, _D), _f32)

    A1 = (We1[:, :_D] - We1[:, _D:]).T
    B1 = We1[:, _D:].T
    A2 = (Wd1[:, :_HID] - Wd1[:, _HID:]).T
    B2 = Wd1[:, _HID:].T

    t1 = pl.pallas_call(
        _node1_body,
        out_shape=jax.ShapeDtypeStruct((_NACC, _D), _f32),
    )(x, gamma.reshape(1, _D), beta.reshape(1, _D), A1, B1, be1.reshape(1, _BIG))

    acc1 = zacc
    for i in range(_NSL):
        gd_i, gs_i = _sc_gather(t1, gds[i], gss[i], _KPQ)
        m_i = _edge_mlp1(gd_i, gs_i, We2.T, be2.reshape(1, _BIG))
        acc1 = _sc_scatter(m_i.reshape(_NW, _KPQ, _CH, _D), sds[i], acc1, _KPQ)

    eps = jax.random.normal(jax.random.key(42), (_N, _HID), _f32)
    mu, lv, t2, cinv = pl.pallas_call(
        _node2_body,
        out_shape=(jax.ShapeDtypeStruct((_N, _HID), _f32),
                   jax.ShapeDtypeStruct((_N, _HID), _f32),
                   jax.ShapeDtypeStruct((_NACC, _D), _f32),
                   jax.ShapeDtypeStruct((_N, 8), _f32)),
    )(acc1, eps, Wmu.T, bmu.reshape(1, _HID), Wvar.T, bvar.reshape(1, _HID),
      A2, bd1.reshape(1, _BIG), B2)

    acc2 = zacc
    for i in range(_NSL):
        gd_i, gs_i = _sc_gather(t2, gds[i], gss[i], _KPQ)
        m_i = _edge_mlp2(gd_i, gs_i, Wd2.T, bd2.reshape(1, _BIG), Wd3.T, bd3.reshape(1, _D))
        acc2 = _sc_scatter(m_i.reshape(_NW, _KPQ, _CH, _D), sds[i], acc2, _KPQ)

    out = pl.pallas_call(
        _out_body,
        out_shape=jax.ShapeDtypeStruct((_N, _D), _f32),
    )(acc2, cinv)

    return (out, mu, lv)


# speedup vs baseline: 5.5257x; 1.0049x over previous
import functools

import jax
import jax.numpy as jnp
from jax import lax
from jax.experimental import pallas as pl
from jax.experimental.pallas import tpu as pltpu
from jax.experimental.pallas import tpu_sc as plsc

_N = 10000
_E = 320000
_D = 128
_BIG = 64
_HID = 32
_NC = 2
_NS = 16
_NW = _NC * _NS
_CH = 128
_KP = 80
_EPT = _KP * _CH
_EP = _NW * _EPT
_NACC = 10240
_RPT = _NACC // _NS
_BE = 2048

_f32 = jnp.float32
_bf16 = jnp.bfloat16


def _node1_body(x_ref, g_ref, b_ref, a1_ref, b1_ref, be1_ref, t_ref):
    x = x_ref[...]
    mean = jnp.mean(x, axis=0, keepdims=True)
    xc = x - mean
    var = jnp.mean(xc * xc, axis=0, keepdims=True)
    xn = xc * lax.rsqrt(var + 1e-5) * g_ref[...] + b_ref[...]
    p = jnp.dot(xn, a1_ref[...], preferred_element_type=_f32) + be1_ref[...]
    q = jnp.dot(xn, b1_ref[...], preferred_element_type=_f32)
    t_ref[:_N, :] = jnp.concatenate([p, q], axis=1)


def _edge1_body(gd_ref, gs_ref, w2_ref, b2_ref, m_ref):
    pre = jnp.maximum(gd_ref[:, :_BIG] + gs_ref[:, _BIG:], 0.0)
    m = jnp.dot(pre, w2_ref[...], preferred_element_type=_f32) + b2_ref[...]
    m = jnp.maximum(m, 0.0)
    m_ref[...] = jnp.concatenate([m, jnp.ones((_BE, _BIG), _f32)], axis=1)


def _edge2_body(gd_ref, gs_ref, w2_ref, b2_ref, w3_ref, b3_ref, m_ref):
    pre = jnp.maximum(gd_ref[:, :_BIG] + gs_ref[:, _BIG:], 0.0)
    t = jnp.dot(pre, w2_ref[...], preferred_element_type=_f32) + b2_ref[...]
    t = jnp.maximum(t, 0.0)
    m_ref[...] = jnp.dot(t, w3_ref[...], preferred_element_type=_f32) + b3_ref[...]


def _node2_body(acc_ref, eps_ref, wmu_ref, bmu_ref, wv_ref, bv_ref,
                a2_ref, bd1_ref, b2_ref,
                mu_ref, lv_ref, t2_ref, ci_ref):
    s = acc_ref[0, :_N, :] + acc_ref[1, :_N, :]
    inv = 1.0 / jnp.maximum(s[:, _BIG:_BIG + 1], 1.0)
    h = s[:, :_BIG] * inv
    mu = jnp.dot(h, wmu_ref[...], preferred_element_type=_f32) + bmu_ref[...]
    lv = jnp.dot(h, wv_ref[...], preferred_element_type=_f32) + bv_ref[...]
    z = mu + eps_ref[...] * jnp.exp(0.5 * lv)
    mu_ref[...] = mu
    lv_ref[...] = lv
    p2 = jnp.dot(z, a2_ref[...], preferred_element_type=_f32) + bd1_ref[...]
    q2 = jnp.dot(z, b2_ref[...], preferred_element_type=_f32)
    t2_ref[:_N, :] = jnp.concatenate([p2, q2], axis=1)
    ci_ref[...] = jnp.broadcast_to(inv, (_N, 8))


def _out_body(acc_ref, ci_ref, o_ref):
    s = acc_ref[0, :_N, :] + acc_ref[1, :_N, :]
    o_ref[...] = s * ci_ref[:, 0:1]


def _sc_gather(table, idx_d, idx_s, kp):
    mesh = plsc.VectorSubcoreMesh(core_axis_name="c", subcore_axis_name="s",
                                  num_cores=_NC, num_subcores=_NS)
    osh = jax.ShapeDtypeStruct((_NW, kp, _CH, _D), _f32)

    @functools.partial(
        pl.kernel,
        out_type=(osh, osh),
        mesh=mesh,
        scratch_types=[
            pltpu.VMEM((kp, _CH), jnp.int32),
            pltpu.VMEM((2, _CH, _D), _f32),
            pltpu.VMEM_SHARED((_NACC, _D), _f32),
            pltpu.SemaphoreType.DMA,
            pltpu.SemaphoreType.DMA,
        ],
    )
    def k(tab, ip, iq, od, os_, ixv, bufs, tabs, gsem, wsem):
        cid = lax.axis_index("c")
        sid = lax.axis_index("s")
        wid = sid * _NC + cid
        r0 = sid * _RPT
        pltpu.sync_copy(tab.at[pl.ds(r0, _RPT)], tabs.at[pl.ds(r0, _RPT)])
        plsc.subcore_barrier()

        def run_pass(ix, out):
            pltpu.sync_copy(ix.at[wid], ixv)
            pltpu.async_copy(tabs.at[ixv.at[0]], bufs.at[0], gsem)

            @pl.loop(0, kp // 2)
            def _(g):
                for b in range(2):
                    j = g * 2 + b
                    pltpu.make_async_copy(tabs.at[ixv.at[j]], bufs.at[b], gsem).wait()
                    pltpu.async_copy(bufs.at[b], out.at[wid, j], wsem)

                    @pl.when(j >= 1)
                    def _():
                        pltpu.make_async_copy(bufs.at[b], out.at[wid, j], wsem).wait()

                    @pl.when(j + 1 < kp)
                    def _():
                        pltpu.async_copy(tabs.at[ixv.at[j + 1]], bufs.at[(b + 1) % 2], gsem)

            pltpu.make_async_copy(bufs.at[0], out.at[wid, 0], wsem).wait()

        run_pass(ip, od)
        run_pass(iq, os_)

    return k(table, idx_d, idx_s)


def _sc_scatter(msgs, idx, init, kp):
    mesh = plsc.VectorSubcoreMesh(core_axis_name="c", subcore_axis_name="s",
                                  num_cores=_NC, num_subcores=_NS)
    zero_init = init is None

    @functools.partial(
        pl.kernel,
        out_type=jax.ShapeDtypeStruct((_NC, _NACC, _D), _f32),
        mesh=mesh,
        scratch_types=[
            pltpu.VMEM((kp, _CH), jnp.int32),
            pltpu.VMEM((2, _CH, _D), _f32),
            pltpu.VMEM_SHARED((_NACC, _D), _f32),
            pltpu.SemaphoreType.DMA,
            pltpu.SemaphoreType.DMA,
        ],
    )
    def k(m, ix, *rest):
        if zero_init:
            out, ixv, bufs, acc, lsem, ssem = rest
        else:
            z, out, ixv, bufs, acc, lsem, ssem = rest
        cid = lax.axis_index("c")
        sid = lax.axis_index("s")
        wid = sid * _NC + cid
        r0 = sid * _RPT
        if zero_init:
            @pl.loop(0, _CH)
            def _(i):
                for c in range(_D // 16):
                    bufs[0, i, pl.ds(c * 16, 16)] = jnp.zeros((16,), _f32)

            for r in range(_RPT // _CH):
                pltpu.sync_copy(bufs.at[0], acc.at[pl.ds(r0 + r * _CH, _CH)])
        else:
            pltpu.sync_copy(z.at[cid, pl.ds(r0, _RPT)], acc.at[pl.ds(r0, _RPT)])
        plsc.subcore_barrier()
        pltpu.sync_copy(ix.at[wid], ixv)

        pltpu.async_copy(m.at[wid, 0], bufs.at[0], lsem)

        @pl.loop(0, kp // 2)
        def _(g):
            for b in range(2):
                j = g * 2 + b
                pltpu.make_async_copy(m.at[wid, j], bufs.at[b], lsem).wait()
                pltpu.async_copy(bufs.at[b], acc.at[ixv.at[j]], ssem, add=True)

                @pl.when(j >= 1)
                def _():
                    pltpu.make_async_copy(bufs.at[b], acc.at[ixv.at[j]], ssem).wait()

                @pl.when(j + 1 < kp)
                def _():
                    pltpu.async_copy(m.at[wid, j + 1], bufs.at[(b + 1) % 2], lsem)

        pltpu.make_async_copy(bufs.at[0], acc.at[ixv.at[0]], ssem).wait()

        plsc.subcore_barrier()
        pltpu.sync_copy(acc.at[pl.ds(r0, _RPT)], out.at[cid, pl.ds(r0, _RPT)])

    return k(msgs, idx) if zero_init else k(msgs, idx, init)


_SLICES = (12, 28, 28, 12)


def _edge_mlp1(gd, gs, w2t, b2r, nrows):
    return pl.pallas_call(
        _edge1_body,
        grid=(nrows // _BE,),
        in_specs=[
            pl.BlockSpec((_BE, _D), lambda i: (i, 0)),
            pl.BlockSpec((_BE, _D), lambda i: (i, 0)),
            pl.BlockSpec((_BIG, _BIG), lambda i: (0, 0)),
            pl.BlockSpec((1, _BIG), lambda i: (0, 0)),
        ],
        out_specs=pl.BlockSpec((_BE, _D), lambda i: (i, 0)),
        out_shape=jax.ShapeDtypeStruct((nrows, _D), _f32),
        compiler_params=pltpu.CompilerParams(
            dimension_semantics=("arbitrary",)),
    )(gd.reshape(nrows, _D), gs.reshape(nrows, _D), w2t, b2r)


def _edge_mlp2(gd, gs, w2t, b2r, w3t, b3r, nrows):
    return pl.pallas_call(
        _edge2_body,
        grid=(nrows // _BE,),
        in_specs=[
            pl.BlockSpec((_BE, _D), lambda i: (i, 0)),
            pl.BlockSpec((_BE, _D), lambda i: (i, 0)),
            pl.BlockSpec((_BIG, _BIG), lambda i: (0, 0)),
            pl.BlockSpec((1, _BIG), lambda i: (0, 0)),
            pl.BlockSpec((_BIG, _D), lambda i: (0, 0)),
            pl.BlockSpec((1, _D), lambda i: (0, 0)),
        ],
        out_specs=pl.BlockSpec((_BE, _D), lambda i: (i, 0)),
        out_shape=jax.ShapeDtypeStruct((nrows, _D), _f32),
        compiler_params=pltpu.CompilerParams(
            dimension_semantics=("arbitrary",)),
    )(gd.reshape(nrows, _D), gs.reshape(nrows, _D), w2t, b2r, w3t, b3r)


def kernel(x, edge_index, gamma, beta, We1, be1, We2, be2, Wmu, bmu,
           Wvar, bvar, Wd1, bd1, Wd2, bd2, Wd3, bd3):
    src = edge_index[0].astype(jnp.int32)
    dst = edge_index[1].astype(jnp.int32)
    pad = _EP - _E
    zpad = jnp.zeros((pad,), jnp.int32)
    g_dst = jnp.concatenate([dst, zpad]).reshape(_NW, _KP, _CH)
    g_src = jnp.concatenate([src, zpad]).reshape(_NW, _KP, _CH)
    s_dst = jnp.concatenate([dst, jnp.full((pad,), _N, jnp.int32)]).reshape(_NW, _KP, _CH)
    offs = [sum(_SLICES[:i]) for i in range(len(_SLICES))]
    gds = [g_dst[:, o:o + k] for o, k in zip(offs, _SLICES)]
    gss = [g_src[:, o:o + k] for o, k in zip(offs, _SLICES)]
    sds = [s_dst[:, o:o + k] for o, k in zip(offs, _SLICES)]

    A1 = (We1[:, :_D] - We1[:, _D:]).T
    B1 = We1[:, _D:].T
    A2 = (Wd1[:, :_HID] - Wd1[:, _HID:]).T
    B2 = Wd1[:, _HID:].T

    t1 = pl.pallas_call(
        _node1_body,
        out_shape=jax.ShapeDtypeStruct((_NACC, _D), _f32),
    )(x, gamma.reshape(1, _D), beta.reshape(1, _D), A1, B1, be1.reshape(1, _BIG))

    acc1 = None
    for i, kp in enumerate(_SLICES):
        gd_i, gs_i = _sc_gather(t1, gds[i], gss[i], kp)
        m_i = _edge_mlp1(gd_i, gs_i, We2.T, be2.reshape(1, _BIG), _NW * kp * _CH)
        acc1 = _sc_scatter(m_i.reshape(_NW, kp, _CH, _D), sds[i], acc1, kp)

    eps = jax.random.normal(jax.random.key(42), (_N, _HID), _f32)
    mu, lv, t2, cinv = pl.pallas_call(
        _node2_body,
        out_shape=(jax.ShapeDtypeStruct((_N, _HID), _f32),
                   jax.ShapeDtypeStruct((_N, _HID), _f32),
                   jax.ShapeDtypeStruct((_NACC, _D), _f32),
                   jax.ShapeDtypeStruct((_N, 8), _f32)),
    )(acc1, eps, Wmu.T, bmu.reshape(1, _HID), Wvar.T, bvar.reshape(1, _HID),
      A2, bd1.reshape(1, _BIG), B2)

    acc2 = None
    for i, kp in enumerate(_SLICES):
        gd_i, gs_i = _sc_gather(t2, gds[i], gss[i], kp)
        m_i = _edge_mlp2(gd_i, gs_i, Wd2.T, bd2.reshape(1, _BIG), Wd3.T, bd3.reshape(1, _D),
                         _NW * kp * _CH)
        acc2 = _sc_scatter(m_i.reshape(_NW, kp, _CH, _D), sds[i], acc2, kp)

    out = pl.pallas_call(
        _out_body,
        out_shape=jax.ShapeDtypeStruct((_N, _D), _f32),
    )(acc2, cinv)

    return (out, mu, lv)
